# baseline (speedup 1.0000x reference)
import functools

import jax
import jax.numpy as jnp
from jax.experimental import pallas as pl
from jax.experimental.pallas import tpu as pltpu

_BM = 1024
_NCHUNKS = 4


def _dot_tb(xv, wv):
    return jax.lax.dot_general(
        xv, wv,
        dimension_numbers=(((1,), (1,)), ((), ())),
        preferred_element_type=jnp.float32)


def _linear_kernel(w_hbm, x_ref, b_ref, o_ref, ws_ref, sems):
    nc = _NCHUNKS
    n_rows = ws_ref.shape[0] // nc

    def _copy(c):
        return pltpu.make_async_copy(
            w_hbm.at[pl.ds(c * n_rows, n_rows), :],
            ws_ref.at[pl.ds(c * n_rows, n_rows), :],
            sems.at[c])

    @pl.when(pl.program_id(1) == 0)
    def _():
        for c in range(nc):
            _copy(c).start()
        for c in range(nc):
            _copy(c).wait()
            cols = slice(c * n_rows, (c + 1) * n_rows)
            o_ref[:, cols] = (
                _dot_tb(x_ref[...], ws_ref[cols, :]) + b_ref[:, cols])

    @pl.when(pl.program_id(1) > 0)
    def _():
        o_ref[...] = _dot_tb(x_ref[...], ws_ref[...]) + b_ref[...]


@functools.partial(jax.jit, static_argnames=("bm",))
def _forward(x, w, b, *, bm):
    M, K = x.shape
    N = w.shape[0]
    b_row = b.reshape(1, N)
    steps = M // bm // 2
    grid = (2, steps)
    out = pl.pallas_call(
        _linear_kernel,
        out_shape=jax.ShapeDtypeStruct((M, N), jnp.float32),
        grid=grid,
        in_specs=[
            pl.BlockSpec(memory_space=pltpu.MemorySpace.HBM),
            pl.BlockSpec((bm, K), lambda i, j: (i * steps + j, 0)),
            pl.BlockSpec((1, N), lambda i, j: (0, 0)),
        ],
        out_specs=pl.BlockSpec((bm, N), lambda i, j: (i * steps + j, 0)),
        scratch_shapes=[
            pltpu.VMEM((N, K), jnp.float32),
            pltpu.SemaphoreType.DMA((_NCHUNKS,)),
        ],
        compiler_params=pltpu.CompilerParams(
            dimension_semantics=("parallel", "arbitrary"),
            vmem_limit_bytes=62 * 1024 * 1024),
        cost_estimate=pl.CostEstimate(
            flops=2 * M * N * K,
            bytes_accessed=4 * M * K + 4 * K * N + 4 * M * N,
            transcendentals=0),
    )(w, x, b_row)
    return out


def kernel(x, w, b):
    bm = _BM if x.shape[0] % (2 * _BM) == 0 else 8
    return _forward(x, w, b, bm=bm)


# speedup vs baseline: 1.0996x; 1.0996x over previous
import functools

import jax
import jax.numpy as jnp
from jax.experimental import pallas as pl
from jax.experimental.pallas import tpu as pltpu

_BM = 1024


def _linear_kernel(w_ref, x_ref, b_ref, o_ref):
    acc = jax.lax.dot_general(
        x_ref[...], w_ref[...],
        dimension_numbers=(((1,), (1,)), ((), ())),
        preferred_element_type=jnp.float32)
    o_ref[...] = acc + b_ref[...]


@functools.partial(jax.jit, static_argnames=("bm",))
def _forward(x, w, b, *, bm):
    M, K = x.shape
    N = w.shape[0]
    b_row = b.reshape(1, N)
    steps = M // bm // 2
    grid = (2, steps)
    out = pl.pallas_call(
        _linear_kernel,
        out_shape=jax.ShapeDtypeStruct((M, N), jnp.float32),
        grid=grid,
        in_specs=[
            pl.BlockSpec((N, K), lambda i, j: (0, 0)),
            pl.BlockSpec((bm, K), lambda i, j: (i * steps + j, 0)),
            pl.BlockSpec((1, N), lambda i, j: (0, 0)),
        ],
        out_specs=pl.BlockSpec((bm, N), lambda i, j: (i * steps + j, 0)),
        compiler_params=pltpu.CompilerParams(
            dimension_semantics=("parallel", "arbitrary"),
            vmem_limit_bytes=62 * 1024 * 1024),
        cost_estimate=pl.CostEstimate(
            flops=2 * M * N * K,
            bytes_accessed=4 * M * K + 4 * K * N + 4 * M * N,
            transcendentals=0),
    )(w, x, b_row)
    return out


def kernel(x, w, b):
    bm = _BM if x.shape[0] % (2 * _BM) == 0 else 8
    return _forward(x, w, b, bm=bm)
